# baseline (device time: 790710 ns/iter reference)
import jax
import jax.numpy as jnp
from jax import lax
from jax.experimental import pallas as pl
from jax.experimental.pallas import tpu as pltpu

N_DEV = 16
M_BLK = 256
N_OUT = 8192
HALF = N_OUT // 2


def kernel(x, w_mat):
    m_glob, k_loc = x.shape
    _, n_out = w_mat.shape
    assert m_glob == N_DEV * M_BLK and n_out == N_OUT

    def body(x_ref, w_ref, out_ref,
             send_a, send_b, recv_a, recv_b,
             send_sem_a, send_sem_b, recv_sem_a, recv_sem_b,
             credit_a, credit_b,
             amax_buf, amax_send_sems, amax_recv_sems):
        p = lax.axis_index("i")
        left = jnp.mod(p - 1 + N_DEV, N_DEV)
        right = jnp.mod(p + 1, N_DEV)

        barrier_sem = pltpu.get_barrier_semaphore()
        for nbr in (left, right):
            pl.semaphore_signal(
                barrier_sem, inc=1,
                device_id=(nbr,), device_id_type=pl.DeviceIdType.MESH,
            )
        pl.semaphore_wait(barrier_sem, 2)

        def blk(c):
            return x_ref[pl.ds(c * M_BLK, M_BLK), :]

        for t in range(N_DEV - 1):
            c_a = jnp.mod(p - 1 - t + 2 * N_DEV, N_DEV)
            c_b = jnp.mod(p + 1 + t, N_DEV)
            acc_a = jnp.dot(blk(c_a), w_ref[:, :HALF],
                            preferred_element_type=jnp.float32)
            acc_b = jnp.dot(blk(c_b), w_ref[:, HALF:],
                            preferred_element_type=jnp.float32)
            if t > 0:
                acc_a = acc_a + recv_a[...]
                acc_b = acc_b + recv_b[...]
            send_a[...] = acc_a
            send_b[...] = acc_b
            if t > 0:
                pl.semaphore_signal(
                    credit_a, inc=1,
                    device_id=(left,), device_id_type=pl.DeviceIdType.MESH,
                )
                pl.semaphore_signal(
                    credit_b, inc=1,
                    device_id=(right,), device_id_type=pl.DeviceIdType.MESH,
                )
                pl.semaphore_wait(credit_a, 1)
                pl.semaphore_wait(credit_b, 1)
            rdma_a = pltpu.make_async_remote_copy(
                src_ref=send_a, dst_ref=recv_a,
                send_sem=send_sem_a, recv_sem=recv_sem_a,
                device_id=(right,), device_id_type=pl.DeviceIdType.MESH,
            )
            rdma_b = pltpu.make_async_remote_copy(
                src_ref=send_b, dst_ref=recv_b,
                send_sem=send_sem_b, recv_sem=recv_sem_b,
                device_id=(left,), device_id_type=pl.DeviceIdType.MESH,
            )
            rdma_a.start()
            rdma_b.start()
            rdma_a.wait()
            rdma_b.wait()

        y_a = jnp.dot(blk(p), w_ref[:, :HALF],
                      preferred_element_type=jnp.float32) + recv_a[...]
        y_b = jnp.dot(blk(p), w_ref[:, HALF:],
                      preferred_element_type=jnp.float32) + recv_b[...]
        out_ref[:, :HALF] = y_a
        out_ref[:, HALF:] = y_b

        local_amax = jnp.maximum(jnp.max(jnp.abs(y_a)), jnp.max(jnp.abs(y_b)))
        amax_buf[0, :, :] = jnp.full((8, 128), local_amax, jnp.float32)
        descs = []
        for d in range(1, N_DEV):
            tgt = jnp.mod(p + d, N_DEV)
            desc = pltpu.make_async_remote_copy(
                src_ref=amax_buf.at[0], dst_ref=amax_buf.at[d],
                send_sem=amax_send_sems.at[d], recv_sem=amax_recv_sems.at[d],
                device_id=(tgt,), device_id_type=pl.DeviceIdType.MESH,
            )
            desc.start()
            descs.append(desc)
        for desc in descs:
            desc.wait_send()
            desc.wait_recv()
        g_amax = jnp.max(amax_buf[:, 0, 0])

        scale = g_amax / 448.0
        y = out_ref[...]
        q = jnp.clip(y / scale, -448.0, 448.0)
        deq = q.astype(jnp.float8_e4m3fn).astype(jnp.float32) * scale
        out_ref[...] = deq

    return pl.pallas_call(
        body,
        out_shape=jax.ShapeDtypeStruct((M_BLK, N_OUT), jnp.float32),
        in_specs=[
            pl.BlockSpec(memory_space=pltpu.VMEM),
            pl.BlockSpec(memory_space=pltpu.VMEM),
        ],
        out_specs=pl.BlockSpec(memory_space=pltpu.VMEM),
        scratch_shapes=[
            pltpu.VMEM((M_BLK, HALF), jnp.float32),
            pltpu.VMEM((M_BLK, HALF), jnp.float32),
            pltpu.VMEM((M_BLK, HALF), jnp.float32),
            pltpu.VMEM((M_BLK, HALF), jnp.float32),
            pltpu.SemaphoreType.DMA,
            pltpu.SemaphoreType.DMA,
            pltpu.SemaphoreType.DMA,
            pltpu.SemaphoreType.DMA,
            pltpu.SemaphoreType.REGULAR,
            pltpu.SemaphoreType.REGULAR,
            pltpu.VMEM((N_DEV, 8, 128), jnp.float32),
            pltpu.SemaphoreType.DMA((N_DEV,)),
            pltpu.SemaphoreType.DMA((N_DEV,)),
        ],
        compiler_params=pltpu.CompilerParams(collective_id=0),
    )(x, w_mat)


# device time: 703896 ns/iter; 1.1233x vs baseline; 1.1233x over previous
import jax
import jax.numpy as jnp
from jax import lax
from jax.experimental import pallas as pl
from jax.experimental.pallas import tpu as pltpu

N_DEV = 16
M_BLK = 256
N_OUT = 8192
HALF = N_OUT // 2
SUBS = 4
SUB = HALF // SUBS


def kernel(x, w_mat):
    m_glob, k_loc = x.shape
    _, n_out = w_mat.shape
    assert m_glob == N_DEV * M_BLK and n_out == N_OUT

    def body(x_ref, w_ref, out_ref,
             send_a, send_b, recv_a, recv_b,
             ssem_a, ssem_b, rsem_a, rsem_b,
             credit_a, credit_b,
             amax_buf, amax_send_sems, amax_recv_sems):
        p = lax.axis_index("i")
        left = jnp.mod(p - 1 + N_DEV, N_DEV)
        right = jnp.mod(p + 1, N_DEV)

        barrier_sem = pltpu.get_barrier_semaphore()
        for nbr in (left, right):
            pl.semaphore_signal(
                barrier_sem, inc=1,
                device_id=(nbr,), device_id_type=pl.DeviceIdType.MESH,
            )
        pl.semaphore_wait(barrier_sem, 2)

        def blk(c):
            return x_ref[pl.ds(c * M_BLK, M_BLK), :]

        def gemm_partials(t):
            c_a = jnp.mod(p - 1 - t + 2 * N_DEV, N_DEV)
            c_b = jnp.mod(p + 1 + t, N_DEV)
            pa = jnp.dot(blk(c_a), w_ref[:, :HALF],
                         preferred_element_type=jnp.float32)
            pb = jnp.dot(blk(c_b), w_ref[:, HALF:],
                         preferred_element_type=jnp.float32)
            return pa, pb

        def make_rdma(t, s, send_buf, recv_buf, ssem, rsem, tgt):
            slot = t % 2
            ds = pl.ds(s * SUB, SUB)
            return pltpu.make_async_remote_copy(
                src_ref=send_buf.at[slot, :, ds],
                dst_ref=recv_buf.at[slot, :, ds],
                send_sem=ssem.at[slot, s],
                recv_sem=rsem.at[slot, s],
                device_id=(tgt,), device_id_type=pl.DeviceIdType.MESH,
            )

        rd_a = {}
        rd_b = {}

        pa, pb = gemm_partials(0)
        for t in range(N_DEV - 1):
            slot = t % 2
            if t >= 2:
                pl.semaphore_wait(credit_a, 1)
                pl.semaphore_wait(credit_b, 1)
            for s in range(SUBS):
                ds = pl.ds(s * SUB, SUB)
                val_a = pa[:, s * SUB:(s + 1) * SUB]
                val_b = pb[:, s * SUB:(s + 1) * SUB]
                if t >= 1:
                    rd_a[(t - 1, s)].wait_recv()
                    rd_b[(t - 1, s)].wait_recv()
                    val_a = val_a + recv_a[(t - 1) % 2, :, ds]
                    val_b = val_b + recv_b[(t - 1) % 2, :, ds]
                if t >= 2:
                    rd_a[(t - 2, s)].wait_send()
                    rd_b[(t - 2, s)].wait_send()
                send_a[slot, :, ds] = val_a
                send_b[slot, :, ds] = val_b
                rd_a[(t, s)] = make_rdma(t, s, send_a, recv_a, ssem_a,
                                         rsem_a, right)
                rd_b[(t, s)] = make_rdma(t, s, send_b, recv_b, ssem_b,
                                         rsem_b, left)
                rd_a[(t, s)].start()
                rd_b[(t, s)].start()
            if 1 <= t <= N_DEV - 3:
                pl.semaphore_signal(
                    credit_a, inc=1,
                    device_id=(left,), device_id_type=pl.DeviceIdType.MESH,
                )
                pl.semaphore_signal(
                    credit_b, inc=1,
                    device_id=(right,), device_id_type=pl.DeviceIdType.MESH,
                )
            pa, pb = gemm_partials(t + 1)

        tl = N_DEV - 2
        for s in range(SUBS):
            ds = pl.ds(s * SUB, SUB)
            rd_a[(tl, s)].wait_recv()
            rd_b[(tl, s)].wait_recv()
            out_ref[:, s * SUB:(s + 1) * SUB] = (
                pa[:, s * SUB:(s + 1) * SUB] + recv_a[tl % 2, :, ds])
            out_ref[:, HALF + s * SUB:HALF + (s + 1) * SUB] = (
                pb[:, s * SUB:(s + 1) * SUB] + recv_b[tl % 2, :, ds])
        for t in (tl - 1, tl):
            for s in range(SUBS):
                rd_a[(t, s)].wait_send()
                rd_b[(t, s)].wait_send()

        local_amax = jnp.float32(0.0)
        for i in range(0, N_OUT, SUB):
            local_amax = jnp.maximum(
                local_amax, jnp.max(jnp.abs(out_ref[:, i:i + SUB])))
        amax_buf[0, :, :] = jnp.full((8, 128), local_amax, jnp.float32)
        descs = []
        for d in range(1, N_DEV):
            tgt = jnp.mod(p + d, N_DEV)
            desc = pltpu.make_async_remote_copy(
                src_ref=amax_buf.at[0], dst_ref=amax_buf.at[d],
                send_sem=amax_send_sems.at[d], recv_sem=amax_recv_sems.at[d],
                device_id=(tgt,), device_id_type=pl.DeviceIdType.MESH,
            )
            desc.start()
            descs.append(desc)
        for desc in descs:
            desc.wait_send()
            desc.wait_recv()
        g_amax = jnp.max(amax_buf[:, 0, 0])

        scale = g_amax / 448.0
        inv_scale = 448.0 / g_amax
        for i in range(0, N_OUT, SUB):
            y = out_ref[:, i:i + SUB]
            q = jnp.clip(y * inv_scale, -448.0, 448.0)
            out_ref[:, i:i + SUB] = (
                q.astype(jnp.float8_e4m3fn).astype(jnp.float32) * scale)

    return pl.pallas_call(
        body,
        out_shape=jax.ShapeDtypeStruct((M_BLK, N_OUT), jnp.float32),
        in_specs=[
            pl.BlockSpec(memory_space=pltpu.VMEM),
            pl.BlockSpec(memory_space=pltpu.VMEM),
        ],
        out_specs=pl.BlockSpec(memory_space=pltpu.VMEM),
        scratch_shapes=[
            pltpu.VMEM((2, M_BLK, HALF), jnp.float32),
            pltpu.VMEM((2, M_BLK, HALF), jnp.float32),
            pltpu.VMEM((2, M_BLK, HALF), jnp.float32),
            pltpu.VMEM((2, M_BLK, HALF), jnp.float32),
            pltpu.SemaphoreType.DMA((2, SUBS)),
            pltpu.SemaphoreType.DMA((2, SUBS)),
            pltpu.SemaphoreType.DMA((2, SUBS)),
            pltpu.SemaphoreType.DMA((2, SUBS)),
            pltpu.SemaphoreType.REGULAR,
            pltpu.SemaphoreType.REGULAR,
            pltpu.VMEM((N_DEV, 8, 128), jnp.float32),
            pltpu.SemaphoreType.DMA((N_DEV,)),
            pltpu.SemaphoreType.DMA((N_DEV,)),
        ],
        compiler_params=pltpu.CompilerParams(
            collective_id=0, vmem_limit_bytes=100 * 1024 * 1024),
    )(x, w_mat)


# device time: 702898 ns/iter; 1.1249x vs baseline; 1.0014x over previous
import jax
import jax.numpy as jnp
from jax import lax
from jax.experimental import pallas as pl
from jax.experimental.pallas import tpu as pltpu

N_DEV = 16
M_BLK = 256
N_OUT = 8192
HALF = N_OUT // 2
SUBS = 8
SUB = HALF // SUBS


def kernel(x, w_mat):
    m_glob, k_loc = x.shape
    _, n_out = w_mat.shape
    assert m_glob == N_DEV * M_BLK and n_out == N_OUT

    def body(x_ref, w_ref, out_ref,
             send_a, send_b, recv_a, recv_b,
             ssem_a, ssem_b, rsem_a, rsem_b,
             credit_a, credit_b,
             amax_buf, amax_send_sems, amax_recv_sems):
        p = lax.axis_index("i")
        left = jnp.mod(p - 1 + N_DEV, N_DEV)
        right = jnp.mod(p + 1, N_DEV)

        barrier_sem = pltpu.get_barrier_semaphore()
        for nbr in (left, right):
            pl.semaphore_signal(
                barrier_sem, inc=1,
                device_id=(nbr,), device_id_type=pl.DeviceIdType.MESH,
            )
        pl.semaphore_wait(barrier_sem, 2)

        def blk(c):
            return x_ref[pl.ds(c * M_BLK, M_BLK), :]

        def gemm_partials(t):
            c_a = jnp.mod(p - 1 - t + 2 * N_DEV, N_DEV)
            c_b = jnp.mod(p + 1 + t, N_DEV)
            pa = jnp.dot(blk(c_a), w_ref[:, :HALF],
                         preferred_element_type=jnp.float32)
            pb = jnp.dot(blk(c_b), w_ref[:, HALF:],
                         preferred_element_type=jnp.float32)
            return pa, pb

        def make_rdma(t, s, send_buf, recv_buf, ssem, rsem, tgt):
            slot = t % 2
            ds = pl.ds(s * SUB, SUB)
            return pltpu.make_async_remote_copy(
                src_ref=send_buf.at[slot, :, ds],
                dst_ref=recv_buf.at[slot, :, ds],
                send_sem=ssem.at[slot, s],
                recv_sem=rsem.at[slot, s],
                device_id=(tgt,), device_id_type=pl.DeviceIdType.MESH,
            )

        rd_a = {}
        rd_b = {}

        c_a0 = jnp.mod(p - 1 + N_DEV, N_DEV)
        c_b0 = jnp.mod(p + 1, N_DEV)
        pa = pb = None
        for t in range(N_DEV - 1):
            slot = t % 2
            if t >= 2:
                pl.semaphore_wait(credit_a, 1)
                pl.semaphore_wait(credit_b, 1)
            for s in range(SUBS):
                ds = pl.ds(s * SUB, SUB)
                if t == 0:
                    val_a = jnp.dot(blk(c_a0), w_ref[:, s * SUB:(s + 1) * SUB],
                                    preferred_element_type=jnp.float32)
                    val_b = jnp.dot(
                        blk(c_b0), w_ref[:, HALF + s * SUB:HALF + (s + 1) * SUB],
                        preferred_element_type=jnp.float32)
                else:
                    val_a = pa[:, s * SUB:(s + 1) * SUB]
                    val_b = pb[:, s * SUB:(s + 1) * SUB]
                    rd_a[(t - 1, s)].wait_recv()
                    rd_b[(t - 1, s)].wait_recv()
                    val_a = val_a + recv_a[(t - 1) % 2, :, ds]
                    val_b = val_b + recv_b[(t - 1) % 2, :, ds]
                if t >= 2:
                    rd_a[(t - 2, s)].wait_send()
                    rd_b[(t - 2, s)].wait_send()
                send_a[slot, :, ds] = val_a
                send_b[slot, :, ds] = val_b
                rd_a[(t, s)] = make_rdma(t, s, send_a, recv_a, ssem_a,
                                         rsem_a, right)
                rd_b[(t, s)] = make_rdma(t, s, send_b, recv_b, ssem_b,
                                         rsem_b, left)
                rd_a[(t, s)].start()
                rd_b[(t, s)].start()
            if 1 <= t <= N_DEV - 3:
                pl.semaphore_signal(
                    credit_a, inc=1,
                    device_id=(left,), device_id_type=pl.DeviceIdType.MESH,
                )
                pl.semaphore_signal(
                    credit_b, inc=1,
                    device_id=(right,), device_id_type=pl.DeviceIdType.MESH,
                )
            pa, pb = gemm_partials(t + 1)

        tl = N_DEV - 2
        local_amax = jnp.float32(0.0)
        for s in range(SUBS):
            ds = pl.ds(s * SUB, SUB)
            rd_a[(tl, s)].wait_recv()
            rd_b[(tl, s)].wait_recv()
            y_a = pa[:, s * SUB:(s + 1) * SUB] + recv_a[tl % 2, :, ds]
            y_b = pb[:, s * SUB:(s + 1) * SUB] + recv_b[tl % 2, :, ds]
            local_amax = jnp.maximum(local_amax, jnp.max(jnp.abs(y_a)))
            local_amax = jnp.maximum(local_amax, jnp.max(jnp.abs(y_b)))
            out_ref[:, s * SUB:(s + 1) * SUB] = y_a
            out_ref[:, HALF + s * SUB:HALF + (s + 1) * SUB] = y_b
        for t in (tl - 1, tl):
            for s in range(SUBS):
                rd_a[(t, s)].wait_send()
                rd_b[(t, s)].wait_send()

        amax_buf[0, :, :] = jnp.full((8, 128), local_amax, jnp.float32)
        descs = []
        for d in range(1, N_DEV):
            tgt = jnp.mod(p + d, N_DEV)
            desc = pltpu.make_async_remote_copy(
                src_ref=amax_buf.at[0], dst_ref=amax_buf.at[d],
                send_sem=amax_send_sems.at[d], recv_sem=amax_recv_sems.at[d],
                device_id=(tgt,), device_id_type=pl.DeviceIdType.MESH,
            )
            desc.start()
            descs.append(desc)
        for desc in descs:
            desc.wait_send()
            desc.wait_recv()
        g_amax = jnp.max(amax_buf[:, 0, 0])

        scale = g_amax / 448.0
        inv_scale = 448.0 / g_amax
        for i in range(0, N_OUT, SUB):
            y = out_ref[:, i:i + SUB]
            q = jnp.clip(y * inv_scale, -448.0, 448.0)
            out_ref[:, i:i + SUB] = (
                q.astype(jnp.float8_e4m3fn).astype(jnp.float32) * scale)

    return pl.pallas_call(
        body,
        out_shape=jax.ShapeDtypeStruct((M_BLK, N_OUT), jnp.float32),
        in_specs=[
            pl.BlockSpec(memory_space=pltpu.VMEM),
            pl.BlockSpec(memory_space=pltpu.VMEM),
        ],
        out_specs=pl.BlockSpec(memory_space=pltpu.VMEM),
        scratch_shapes=[
            pltpu.VMEM((2, M_BLK, HALF), jnp.float32),
            pltpu.VMEM((2, M_BLK, HALF), jnp.float32),
            pltpu.VMEM((2, M_BLK, HALF), jnp.float32),
            pltpu.VMEM((2, M_BLK, HALF), jnp.float32),
            pltpu.SemaphoreType.DMA((2, SUBS)),
            pltpu.SemaphoreType.DMA((2, SUBS)),
            pltpu.SemaphoreType.DMA((2, SUBS)),
            pltpu.SemaphoreType.DMA((2, SUBS)),
            pltpu.SemaphoreType.REGULAR,
            pltpu.SemaphoreType.REGULAR,
            pltpu.VMEM((N_DEV, 8, 128), jnp.float32),
            pltpu.SemaphoreType.DMA((N_DEV,)),
            pltpu.SemaphoreType.DMA((N_DEV,)),
        ],
        compiler_params=pltpu.CompilerParams(
            collective_id=0, vmem_limit_bytes=100 * 1024 * 1024),
    )(x, w_mat)


# device time: 702822 ns/iter; 1.1251x vs baseline; 1.0001x over previous
import jax
import jax.numpy as jnp
from jax import lax
from jax.experimental import pallas as pl
from jax.experimental.pallas import tpu as pltpu

N_DEV = 16
M_BLK = 256
N_OUT = 8192
HALF = N_OUT // 2
SUBS = 8
SUB = HALF // SUBS


def kernel(x, w_mat):
    m_glob, k_loc = x.shape
    _, n_out = w_mat.shape
    assert m_glob == N_DEV * M_BLK and n_out == N_OUT

    def body(x_ref, w_ref, out_ref,
             send_a, send_b, recv_a, recv_b,
             ssem_a, ssem_b, rsem_a, rsem_b,
             credit_a, credit_b,
             amax_buf, amax_send_sems, amax_recv_sems):
        p = lax.axis_index("i")
        left = jnp.mod(p - 1 + N_DEV, N_DEV)
        right = jnp.mod(p + 1, N_DEV)

        barrier_sem = pltpu.get_barrier_semaphore()
        for nbr in (left, right):
            pl.semaphore_signal(
                barrier_sem, inc=1,
                device_id=(nbr,), device_id_type=pl.DeviceIdType.MESH,
            )
        pl.semaphore_wait(barrier_sem, 2)

        def blk(c):
            return x_ref[pl.ds(c * M_BLK, M_BLK), :]

        def gemm_partials(t):
            c_a = jnp.mod(p - 1 - t + 2 * N_DEV, N_DEV)
            c_b = jnp.mod(p + 1 + t, N_DEV)
            pa = jnp.dot(blk(c_a), w_ref[:, :HALF],
                         preferred_element_type=jnp.float32)
            pb = jnp.dot(blk(c_b), w_ref[:, HALF:],
                         preferred_element_type=jnp.float32)
            return pa, pb

        def make_rdma(t, s, send_buf, recv_buf, ssem, rsem, tgt):
            slot = t % 2
            ds = pl.ds(s * SUB, SUB)
            return pltpu.make_async_remote_copy(
                src_ref=send_buf.at[slot, :, ds],
                dst_ref=recv_buf.at[slot, :, ds],
                send_sem=ssem.at[slot, s],
                recv_sem=rsem.at[slot, s],
                device_id=(tgt,), device_id_type=pl.DeviceIdType.MESH,
            )

        rd_a = {}
        rd_b = {}

        c_a0 = jnp.mod(p - 1 + N_DEV, N_DEV)
        c_b0 = jnp.mod(p + 1, N_DEV)
        pa = pb = None
        for t in range(N_DEV - 1):
            slot = t % 2
            if t >= 2:
                pl.semaphore_wait(credit_a, 1)
                pl.semaphore_wait(credit_b, 1)
            for s in range(SUBS):
                ds = pl.ds(s * SUB, SUB)
                if t == 0:
                    val_a = jnp.dot(blk(c_a0), w_ref[:, s * SUB:(s + 1) * SUB],
                                    preferred_element_type=jnp.float32)
                    val_b = jnp.dot(
                        blk(c_b0), w_ref[:, HALF + s * SUB:HALF + (s + 1) * SUB],
                        preferred_element_type=jnp.float32)
                else:
                    val_a = pa[:, s * SUB:(s + 1) * SUB]
                    val_b = pb[:, s * SUB:(s + 1) * SUB]
                    rd_a[(t - 1, s)].wait_recv()
                    rd_b[(t - 1, s)].wait_recv()
                    val_a = val_a + recv_a[(t - 1) % 2, :, ds]
                    val_b = val_b + recv_b[(t - 1) % 2, :, ds]
                if t >= 2:
                    rd_a[(t - 2, s)].wait_send()
                    rd_b[(t - 2, s)].wait_send()
                send_a[slot, :, ds] = val_a
                send_b[slot, :, ds] = val_b
                rd_a[(t, s)] = make_rdma(t, s, send_a, recv_a, ssem_a,
                                         rsem_a, right)
                rd_b[(t, s)] = make_rdma(t, s, send_b, recv_b, ssem_b,
                                         rsem_b, left)
                rd_a[(t, s)].start()
                rd_b[(t, s)].start()
            if 1 <= t <= N_DEV - 3:
                pl.semaphore_signal(
                    credit_a, inc=1,
                    device_id=(left,), device_id_type=pl.DeviceIdType.MESH,
                )
                pl.semaphore_signal(
                    credit_b, inc=1,
                    device_id=(right,), device_id_type=pl.DeviceIdType.MESH,
                )
            pa, pb = gemm_partials(t + 1)

        tl = N_DEV - 2
        local_amax = jnp.float32(0.0)
        for s in range(SUBS):
            ds = pl.ds(s * SUB, SUB)
            rd_a[(tl, s)].wait_recv()
            rd_b[(tl, s)].wait_recv()
            y_a = pa[:, s * SUB:(s + 1) * SUB] + recv_a[tl % 2, :, ds]
            y_b = pb[:, s * SUB:(s + 1) * SUB] + recv_b[tl % 2, :, ds]
            local_amax = jnp.maximum(local_amax, jnp.max(jnp.abs(y_a)))
            local_amax = jnp.maximum(local_amax, jnp.max(jnp.abs(y_b)))
            out_ref[:, s * SUB:(s + 1) * SUB] = y_a
            out_ref[:, HALF + s * SUB:HALF + (s + 1) * SUB] = y_b
        amax_buf[0, :, :] = jnp.full((8, 128), local_amax, jnp.float32)
        descs = []
        for d in range(1, N_DEV):
            tgt = jnp.mod(p + d, N_DEV)
            desc = pltpu.make_async_remote_copy(
                src_ref=amax_buf.at[0], dst_ref=amax_buf.at[d],
                send_sem=amax_send_sems.at[d], recv_sem=amax_recv_sems.at[d],
                device_id=(tgt,), device_id_type=pl.DeviceIdType.MESH,
            )
            desc.start()
            descs.append(desc)
        for t in (tl - 1, tl):
            for s in range(SUBS):
                rd_a[(t, s)].wait_send()
                rd_b[(t, s)].wait_send()
        for desc in descs:
            desc.wait_send()
            desc.wait_recv()
        g_amax = jnp.max(amax_buf[:, 0, 0])

        scale = g_amax / 448.0
        inv_scale = 448.0 / g_amax
        for i in range(0, N_OUT, SUB):
            y = out_ref[:, i:i + SUB]
            out_ref[:, i:i + SUB] = (
                (y * inv_scale).astype(jnp.float8_e4m3fn).astype(jnp.float32)
                * scale)

    return pl.pallas_call(
        body,
        out_shape=jax.ShapeDtypeStruct((M_BLK, N_OUT), jnp.float32),
        in_specs=[
            pl.BlockSpec(memory_space=pltpu.VMEM),
            pl.BlockSpec(memory_space=pltpu.VMEM),
        ],
        out_specs=pl.BlockSpec(memory_space=pltpu.VMEM),
        scratch_shapes=[
            pltpu.VMEM((2, M_BLK, HALF), jnp.float32),
            pltpu.VMEM((2, M_BLK, HALF), jnp.float32),
            pltpu.VMEM((2, M_BLK, HALF), jnp.float32),
            pltpu.VMEM((2, M_BLK, HALF), jnp.float32),
            pltpu.SemaphoreType.DMA((2, SUBS)),
            pltpu.SemaphoreType.DMA((2, SUBS)),
            pltpu.SemaphoreType.DMA((2, SUBS)),
            pltpu.SemaphoreType.DMA((2, SUBS)),
            pltpu.SemaphoreType.REGULAR,
            pltpu.SemaphoreType.REGULAR,
            pltpu.VMEM((N_DEV, 8, 128), jnp.float32),
            pltpu.SemaphoreType.DMA((N_DEV,)),
            pltpu.SemaphoreType.DMA((N_DEV,)),
        ],
        compiler_params=pltpu.CompilerParams(
            collective_id=0, vmem_limit_bytes=100 * 1024 * 1024),
    )(x, w_mat)
